# baseline (speedup 1.0000x reference)
import functools

import jax
import jax.numpy as jnp
import numpy as np
from jax import lax
from jax.experimental import pallas as pl
from jax.experimental.pallas import tpu as pltpu

_K1 = (np.uint32(1832780943), np.uint32(270669613))
_K2B = (np.uint32(2350016172), np.uint32(1168365246))
_K3 = (np.uint32(2465931498), np.uint32(255383827))

_TINY = np.float32(np.finfo(np.float32).tiny)
_HI = jax.lax.Precision.HIGHEST


def _threefry_bits(k0, k1, x1):
    rot0 = (13, 15, 26, 6)
    rot1 = (17, 29, 16, 24)
    ks0 = k0
    ks1 = k1
    ks2 = np.uint32(int(k0) ^ int(k1) ^ 0x1BD11BDA)

    def rotl(v, d):
        return lax.shift_left(v, np.uint32(d)) | lax.shift_right_logical(
            v, np.uint32(32 - d))

    def rounds(a, b, rots):
        for r in rots:
            a = a + b
            b = rotl(b, r)
            b = a ^ b
        return a, b

    a = jnp.full_like(x1, ks0)
    b = x1 + ks1
    a, b = rounds(a, b, rot0)
    a = a + ks1
    b = b + np.uint32(int(ks2) + 1 & 0xFFFFFFFF)
    a, b = rounds(a, b, rot1)
    a = a + ks2
    b = b + np.uint32(int(ks0) + 2 & 0xFFFFFFFF)
    a, b = rounds(a, b, rot0)
    a = a + ks0
    b = b + np.uint32(int(ks1) + 3 & 0xFFFFFFFF)
    a, b = rounds(a, b, rot1)
    a = a + ks1
    b = b + np.uint32(int(ks2) + 4 & 0xFFFFFFFF)
    a, b = rounds(a, b, rot0)
    a = a + ks2
    b = b + np.uint32(int(ks0) + 5 & 0xFFFFFFFF)
    return a ^ b


def _gumbel_from_bits(bits):
    fb = lax.shift_right_logical(bits, np.uint32(9)) | np.uint32(0x3F800000)
    f = lax.bitcast_convert_type(fb, jnp.float32) - np.float32(1.0)
    u = jnp.maximum(_TINY, f + _TINY)
    return -jnp.log(-jnp.log(u))


def _u32_iota(shape, dim):
    return lax.broadcasted_iota(jnp.int32, shape, dim).astype(jnp.uint32)


def _first_argmax(vals, ncols):
    del ncols
    idx = jnp.argmax(vals, axis=1)[:, None]
    m = jnp.max(vals, axis=1, keepdims=True)
    return idx, m


def _sampler_kernel(nrows, q_ref, op_ref, nte_ref, se_ref,
                    aug_ref, sc_ref, lp_ref):
    i = pl.program_id(0)
    R = nrows
    b0u = lax.convert_element_type(i, jnp.uint32) * np.uint32(R)

    q = q_ref[...]
    op = op_ref[...]
    nte = nte_ref[...]
    se = se_ref[...]

    dn = (((1,), (1,)), ((), ()))
    nt_row = lax.dot_general(q, nte, dn)
    table = lax.dot_general(op + q, se, dn)
    m_t = jnp.max(table, axis=1, keepdims=True)
    lse_t = m_t + jnp.log(
        jnp.sum(jnp.exp(table - m_t), axis=1, keepdims=True))
    m_nt = jnp.max(nt_row, axis=1, keepdims=True)
    lse_nt = m_nt + jnp.log(
        jnp.sum(jnp.exp(nt_row - m_nt), axis=1, keepdims=True))
    nt_col = lax.transpose(nt_row, (1, 0))

    p_nt = (np.uint32(5) * (_u32_iota((5, R), 1) + b0u)
            + _u32_iota((5, R), 0))
    g_nt = _gumbel_from_bits(_threefry_bits(*_K1, p_nt))
    vals_nt = g_nt + nt_col
    m_vn = jnp.max(vals_nt, axis=0, keepdims=True)
    rows5 = lax.broadcasted_iota(jnp.int32, (5, R), 0)
    sidx_t = jnp.min(jnp.where(vals_nt == m_vn, rows5, 5),
                     axis=0, keepdims=True)
    nt_sel_t = jnp.sum(jnp.where(rows5 == sidx_t, nt_col + jnp.zeros_like(g_nt),
                                 0.0), axis=0, keepdims=True)

    p_ri = (np.uint32(4) * (_u32_iota((4, R), 1) + b0u)
            + _u32_iota((4, R), 0))
    bits_ri = _threefry_bits(*_K2B, p_ri)
    aug_raw_t = (bits_ri & np.uint32(63)).astype(jnp.int32)
    rows4 = lax.broadcasted_iota(jnp.int32, (4, R), 0)
    aug_t = jnp.where(rows4 >= sidx_t, 0, aug_raw_t)

    aug = lax.transpose(aug_t, (1, 0))
    sidx = lax.transpose(sidx_t, (1, 0))
    nt_sel = lax.transpose(nt_sel_t, (1, 0))
    cols4 = lax.broadcasted_iota(jnp.int32, (R, 4), 1)
    mask = cols4 >= sidx
    aug_ref[...] = aug

    cols64 = lax.broadcasted_iota(jnp.int32, (R, 64), 1)
    p0 = (np.uint32(512) * (_u32_iota((R, 128), 0) + b0u)
          + _u32_iota((R, 128), 1))
    table_lse = jnp.concatenate([table, lse_t], axis=1)
    scale_cols = []
    lp_sum = jnp.zeros((R, 1), jnp.float32)
    for l in range(4):
        p_sc = p0 + np.uint32(128 * l) if l else p0
        g_sc = _gumbel_from_bits(_threefry_bits(*_K3, p_sc))
        aug_l = aug[:, l:l + 1]
        oh = (cols64 == aug_l).astype(jnp.float32)
        row_full = lax.dot_general(oh, table_lse, (((1,), (0,)), ((), ())),
                                   precision=_HI)
        row_logits = row_full[:, :128]
        lse_sel = row_full[:, 128:129]
        vals = g_sc + row_logits
        s_l, _ = _first_argmax(vals, 128)
        scale_cols.append(s_l)
        cols128 = lax.broadcasted_iota(jnp.int32, (R, 128), 1)
        sel_logit = jnp.sum(jnp.where(cols128 == s_l, row_logits, 0.0),
                            axis=1, keepdims=True)
        lp_l = sel_logit - lse_sel
        mask_l = mask[:, l:l + 1]
        lp_sum = lp_sum + jnp.where(mask_l, 0.0, lp_l)

    sc_ref[...] = jnp.concatenate(scale_cols, axis=1)

    lp_ref[...] = nt_sel - lse_nt + lp_sum


def kernel(imgs, q_param, op_embs, num_transforms_embs, scale_embs):
    B = imgs.shape[0]
    R = 512
    grid = (B // R,)
    out_shapes = (
        jax.ShapeDtypeStruct((B, 4), jnp.int32),
        jax.ShapeDtypeStruct((B, 4), jnp.int32),
        jax.ShapeDtypeStruct((B, 1), jnp.float32),
    )
    full = lambda shape: pl.BlockSpec(shape, lambda i: (0, 0))
    aug, scales, logps = pl.pallas_call(
        functools.partial(_sampler_kernel, R),
        grid=grid,
        in_specs=[
            full((1, 128)),
            full((64, 128)),
            full((5, 128)),
            full((128, 128)),
        ],
        out_specs=(
            pl.BlockSpec((R, 4), lambda i: (i, 0)),
            pl.BlockSpec((R, 4), lambda i: (i, 0)),
            pl.BlockSpec((R, 1), lambda i: (i, 0)),
        ),
        out_shape=out_shapes,
        compiler_params=pltpu.CompilerParams(
            dimension_semantics=("parallel",),
        ),
    )(q_param.reshape(1, 128), op_embs, num_transforms_embs, scale_embs)
    return aug, scales, logps.reshape(B)


# speedup vs baseline: 1.1019x; 1.1019x over previous
import functools

import jax
import jax.numpy as jnp
import numpy as np
from jax import lax
from jax.experimental import pallas as pl
from jax.experimental.pallas import tpu as pltpu

_K1 = (np.uint32(1832780943), np.uint32(270669613))
_K2B = (np.uint32(2350016172), np.uint32(1168365246))
_K3 = (np.uint32(2465931498), np.uint32(255383827))

_TINY = np.float32(np.finfo(np.float32).tiny)
_HI = jax.lax.Precision.HIGHEST


def _threefry_bits(k0, k1, x1):
    rot0 = (13, 15, 26, 6)
    rot1 = (17, 29, 16, 24)
    ks0 = k0
    ks1 = k1
    ks2 = np.uint32(int(k0) ^ int(k1) ^ 0x1BD11BDA)

    def rotl(v, d):
        return lax.shift_left(v, np.uint32(d)) | lax.shift_right_logical(
            v, np.uint32(32 - d))

    def rounds(a, b, rots):
        for r in rots:
            a = a + b
            b = rotl(b, r)
            b = a ^ b
        return a, b

    a = jnp.full_like(x1, ks0)
    b = x1 + ks1
    a, b = rounds(a, b, rot0)
    a = a + ks1
    b = b + np.uint32(int(ks2) + 1 & 0xFFFFFFFF)
    a, b = rounds(a, b, rot1)
    a = a + ks2
    b = b + np.uint32(int(ks0) + 2 & 0xFFFFFFFF)
    a, b = rounds(a, b, rot0)
    a = a + ks0
    b = b + np.uint32(int(ks1) + 3 & 0xFFFFFFFF)
    a, b = rounds(a, b, rot1)
    a = a + ks1
    b = b + np.uint32(int(ks2) + 4 & 0xFFFFFFFF)
    a, b = rounds(a, b, rot0)
    a = a + ks2
    b = b + np.uint32(int(ks0) + 5 & 0xFFFFFFFF)
    return a ^ b


def _gumbel_from_bits(bits):
    fb = lax.shift_right_logical(bits, np.uint32(9)) | np.uint32(0x3F800000)
    f = lax.bitcast_convert_type(fb, jnp.float32) - np.float32(1.0)
    u = jnp.maximum(_TINY, f + _TINY)
    return -jnp.log(-jnp.log(u))


def _u32_iota(shape, dim):
    return lax.broadcasted_iota(jnp.int32, shape, dim).astype(jnp.uint32)


def _first_argmax(vals, ncols):
    del ncols
    idx = jnp.argmax(vals, axis=1)[:, None]
    m = jnp.max(vals, axis=1, keepdims=True)
    return idx, m


def _sampler_kernel(nrows, q_ref, op_ref, nte_ref, se_ref,
                    aug_ref, sc_ref, lp_ref):
    i = pl.program_id(0)
    R = nrows
    b0u = lax.convert_element_type(i, jnp.uint32) * np.uint32(R)

    q = q_ref[...]
    op = op_ref[...]
    nte = nte_ref[...]
    se = se_ref[...]

    dn = (((1,), (1,)), ((), ()))
    nt_row = lax.dot_general(q, nte, dn)
    table = lax.dot_general(op + q, se, dn)
    m_t = jnp.max(table, axis=1, keepdims=True)
    lse_t = m_t + jnp.log(
        jnp.sum(jnp.exp(table - m_t), axis=1, keepdims=True))
    m_nt = jnp.max(nt_row, axis=1, keepdims=True)
    lse_nt = m_nt + jnp.log(
        jnp.sum(jnp.exp(nt_row - m_nt), axis=1, keepdims=True))
    nt_col = lax.transpose(nt_row, (1, 0))

    p_nt = (np.uint32(5) * (_u32_iota((5, R), 1) + b0u)
            + _u32_iota((5, R), 0))
    g_nt = _gumbel_from_bits(_threefry_bits(*_K1, p_nt))
    vals_nt = g_nt + nt_col
    m_vn = jnp.max(vals_nt, axis=0, keepdims=True)
    rows5 = lax.broadcasted_iota(jnp.int32, (5, R), 0)
    sidx_t = jnp.min(jnp.where(vals_nt == m_vn, rows5, 5),
                     axis=0, keepdims=True)
    nt_sel_t = jnp.sum(jnp.where(rows5 == sidx_t, nt_col + jnp.zeros_like(g_nt),
                                 0.0), axis=0, keepdims=True)

    p_ri = (np.uint32(4) * (_u32_iota((4, R), 1) + b0u)
            + _u32_iota((4, R), 0))
    bits_ri = _threefry_bits(*_K2B, p_ri)
    aug_raw_t = (bits_ri & np.uint32(63)).astype(jnp.int32)
    rows4 = lax.broadcasted_iota(jnp.int32, (4, R), 0)
    aug_t = jnp.where(rows4 >= sidx_t, 0, aug_raw_t)

    aug = lax.transpose(aug_t, (1, 0))
    sidx = lax.transpose(sidx_t, (1, 0))
    nt_sel = lax.transpose(nt_sel_t, (1, 0))
    cols4 = lax.broadcasted_iota(jnp.int32, (R, 4), 1)
    mask = cols4 >= sidx
    aug_ref[...] = aug

    cols64 = lax.broadcasted_iota(jnp.int32, (R, 64), 1)
    p0 = (np.uint32(512) * (_u32_iota((R, 128), 0) + b0u)
          + _u32_iota((R, 128), 1))
    scale_cols = []
    lp_sum = jnp.zeros((R, 1), jnp.float32)
    for l in range(4):
        p_sc = p0 + np.uint32(128 * l) if l else p0
        g_sc = _gumbel_from_bits(_threefry_bits(*_K3, p_sc))
        aug_l = aug[:, l:l + 1]
        oh = (cols64 == aug_l).astype(jnp.float32)
        row_logits = lax.dot_general(oh, table, (((1,), (0,)), ((), ())),
                                     precision=_HI)
        lse_sel = lax.dot_general(oh, lse_t, (((1,), (0,)), ((), ())),
                                  precision=_HI)
        vals = g_sc + row_logits
        s_l, _ = _first_argmax(vals, 128)
        scale_cols.append(s_l)
        cols128 = lax.broadcasted_iota(jnp.int32, (R, 128), 1)
        sel_logit = jnp.sum(jnp.where(cols128 == s_l, row_logits, 0.0),
                            axis=1, keepdims=True)
        lp_l = sel_logit - lse_sel
        mask_l = mask[:, l:l + 1]
        lp_sum = lp_sum + jnp.where(mask_l, 0.0, lp_l)

    sc_ref[...] = jnp.concatenate(scale_cols, axis=1)

    lp_ref[...] = nt_sel - lse_nt + lp_sum


def kernel(imgs, q_param, op_embs, num_transforms_embs, scale_embs):
    B = imgs.shape[0]
    R = 512
    grid = (B // R,)
    out_shapes = (
        jax.ShapeDtypeStruct((B, 4), jnp.int32),
        jax.ShapeDtypeStruct((B, 4), jnp.int32),
        jax.ShapeDtypeStruct((B, 1), jnp.float32),
    )
    full = lambda shape: pl.BlockSpec(shape, lambda i: (0, 0))
    aug, scales, logps = pl.pallas_call(
        functools.partial(_sampler_kernel, R),
        grid=grid,
        in_specs=[
            full((1, 128)),
            full((64, 128)),
            full((5, 128)),
            full((128, 128)),
        ],
        out_specs=(
            pl.BlockSpec((R, 4), lambda i: (i, 0)),
            pl.BlockSpec((R, 4), lambda i: (i, 0)),
            pl.BlockSpec((R, 1), lambda i: (i, 0)),
        ),
        out_shape=out_shapes,
        compiler_params=pltpu.CompilerParams(
            dimension_semantics=("parallel",),
        ),
    )(q_param.reshape(1, 128), op_embs, num_transforms_embs, scale_embs)
    return aug, scales, logps.reshape(B)


# speedup vs baseline: 1.1120x; 1.0092x over previous
import functools

import jax
import jax.numpy as jnp
import numpy as np
from jax import lax
from jax.experimental import pallas as pl
from jax.experimental.pallas import tpu as pltpu

_K1 = (np.uint32(1832780943), np.uint32(270669613))
_K2B = (np.uint32(2350016172), np.uint32(1168365246))
_K3 = (np.uint32(2465931498), np.uint32(255383827))

_TINY = np.float32(np.finfo(np.float32).tiny)
_HI = jax.lax.Precision.HIGHEST


def _threefry_bits(k0, k1, x1):
    rot0 = (13, 15, 26, 6)
    rot1 = (17, 29, 16, 24)
    ks0 = k0
    ks1 = k1
    ks2 = np.uint32(int(k0) ^ int(k1) ^ 0x1BD11BDA)

    def rotl(v, d):
        return lax.shift_left(v, np.uint32(d)) | lax.shift_right_logical(
            v, np.uint32(32 - d))

    def rounds(a, b, rots):
        for r in rots:
            a = a + b
            b = rotl(b, r)
            b = a ^ b
        return a, b

    a = jnp.full_like(x1, ks0)
    b = x1 + ks1
    a, b = rounds(a, b, rot0)
    a = a + ks1
    b = b + np.uint32(int(ks2) + 1 & 0xFFFFFFFF)
    a, b = rounds(a, b, rot1)
    a = a + ks2
    b = b + np.uint32(int(ks0) + 2 & 0xFFFFFFFF)
    a, b = rounds(a, b, rot0)
    a = a + ks0
    b = b + np.uint32(int(ks1) + 3 & 0xFFFFFFFF)
    a, b = rounds(a, b, rot1)
    a = a + ks1
    b = b + np.uint32(int(ks2) + 4 & 0xFFFFFFFF)
    a, b = rounds(a, b, rot0)
    a = a + ks2
    b = b + np.uint32(int(ks0) + 5 & 0xFFFFFFFF)
    return a ^ b


def _gumbel_from_bits(bits):
    fb = lax.shift_right_logical(bits, np.uint32(9)) | np.uint32(0x3F800000)
    f = lax.bitcast_convert_type(fb, jnp.float32) - np.float32(1.0)
    u = f + _TINY
    return -jnp.log(-jnp.log(u))


def _u32_iota(shape, dim):
    return lax.broadcasted_iota(jnp.int32, shape, dim).astype(jnp.uint32)


def _first_argmax(vals, ncols):
    del ncols
    idx = jnp.argmax(vals, axis=1)[:, None]
    m = jnp.max(vals, axis=1, keepdims=True)
    return idx, m


def _sampler_kernel(nrows, q_ref, op_ref, nte_ref, se_ref,
                    aug_ref, sc_ref, lp_ref):
    i = pl.program_id(0)
    R = nrows
    b0u = lax.convert_element_type(i, jnp.uint32) * np.uint32(R)

    q = q_ref[...]
    op = op_ref[...]
    nte = nte_ref[...]
    se = se_ref[...]

    dn = (((1,), (1,)), ((), ()))
    nt_row = lax.dot_general(q, nte, dn)
    table = lax.dot_general(op + q, se, dn)
    m_t = jnp.max(table, axis=1, keepdims=True)
    lse_t = m_t + jnp.log(
        jnp.sum(jnp.exp(table - m_t), axis=1, keepdims=True))
    m_nt = jnp.max(nt_row, axis=1, keepdims=True)
    lse_nt = m_nt + jnp.log(
        jnp.sum(jnp.exp(nt_row - m_nt), axis=1, keepdims=True))
    nt_col = lax.transpose(nt_row, (1, 0))

    p_nt = (np.uint32(5) * (_u32_iota((5, R), 1) + b0u)
            + _u32_iota((5, R), 0))
    g_nt = _gumbel_from_bits(_threefry_bits(*_K1, p_nt))
    vals_nt = g_nt + nt_col
    m_vn = jnp.max(vals_nt, axis=0, keepdims=True)
    rows5 = lax.broadcasted_iota(jnp.int32, (5, R), 0)
    sidx_t = jnp.min(jnp.where(vals_nt == m_vn, rows5, 5),
                     axis=0, keepdims=True)
    nt_sel_t = jnp.sum(jnp.where(rows5 == sidx_t, nt_col + jnp.zeros_like(g_nt),
                                 0.0), axis=0, keepdims=True)

    p_ri = (np.uint32(4) * (_u32_iota((4, R), 1) + b0u)
            + _u32_iota((4, R), 0))
    bits_ri = _threefry_bits(*_K2B, p_ri)
    aug_raw_t = (bits_ri & np.uint32(63)).astype(jnp.int32)
    rows4 = lax.broadcasted_iota(jnp.int32, (4, R), 0)
    aug_t = jnp.where(rows4 >= sidx_t, 0, aug_raw_t)

    aug = lax.transpose(aug_t, (1, 0))
    sidx = lax.transpose(sidx_t, (1, 0))
    nt_sel = lax.transpose(nt_sel_t, (1, 0))
    cols4 = lax.broadcasted_iota(jnp.int32, (R, 4), 1)
    mask = cols4 >= sidx
    aug_ref[...] = aug

    cols64 = lax.broadcasted_iota(jnp.int32, (R, 64), 1)
    p0 = (np.uint32(512) * (_u32_iota((R, 128), 0) + b0u)
          + _u32_iota((R, 128), 1))
    scale_cols = []
    lp_sum = jnp.zeros((R, 1), jnp.float32)
    for l in range(4):
        p_sc = p0 + np.uint32(128 * l) if l else p0
        g_sc = _gumbel_from_bits(_threefry_bits(*_K3, p_sc))
        aug_l = aug[:, l:l + 1]
        oh = (cols64 == aug_l).astype(jnp.float32)
        row_logits = lax.dot_general(oh, table, (((1,), (0,)), ((), ())),
                                     precision=_HI)
        lse_sel = lax.dot_general(oh, lse_t, (((1,), (0,)), ((), ())),
                                  precision=_HI)
        vals = g_sc + row_logits
        s_l, _ = _first_argmax(vals, 128)
        scale_cols.append(s_l)
        cols128 = lax.broadcasted_iota(jnp.int32, (R, 128), 1)
        sel_logit = jnp.sum(jnp.where(cols128 == s_l, row_logits, 0.0),
                            axis=1, keepdims=True)
        lp_l = sel_logit - lse_sel
        mask_l = mask[:, l:l + 1]
        lp_sum = lp_sum + jnp.where(mask_l, 0.0, lp_l)

    sc_ref[...] = jnp.concatenate(scale_cols, axis=1)

    lp_ref[...] = nt_sel - lse_nt + lp_sum


def kernel(imgs, q_param, op_embs, num_transforms_embs, scale_embs):
    B = imgs.shape[0]
    R = 512
    grid = (B // R,)
    out_shapes = (
        jax.ShapeDtypeStruct((B, 4), jnp.int32),
        jax.ShapeDtypeStruct((B, 4), jnp.int32),
        jax.ShapeDtypeStruct((B, 1), jnp.float32),
    )
    full = lambda shape: pl.BlockSpec(shape, lambda i: (0, 0))
    aug, scales, logps = pl.pallas_call(
        functools.partial(_sampler_kernel, R),
        grid=grid,
        in_specs=[
            full((1, 128)),
            full((64, 128)),
            full((5, 128)),
            full((128, 128)),
        ],
        out_specs=(
            pl.BlockSpec((R, 4), lambda i: (i, 0)),
            pl.BlockSpec((R, 4), lambda i: (i, 0)),
            pl.BlockSpec((R, 1), lambda i: (i, 0)),
        ),
        out_shape=out_shapes,
        compiler_params=pltpu.CompilerParams(
            dimension_semantics=("parallel",),
        ),
    )(q_param.reshape(1, 128), op_embs, num_transforms_embs, scale_embs)
    return aug, scales, logps.reshape(B)
